# Initial kernel scaffold; baseline (speedup 1.0000x reference)
import jax
import jax.numpy as jnp
from jax.experimental import pallas as pl


def kernel(x):
    raise NotImplementedError("write your pallas kernel here")


# speedup vs baseline: 12.0419x; 12.0419x over previous
import functools

import jax
import jax.numpy as jnp
import numpy as np
from jax.experimental import pallas as pl
from jax.experimental.pallas import tpu as pltpu

_TOPK = 256


def _f32_sort_key(x):
    s = jax.lax.bitcast_convert_type(x, jnp.int32)
    return jnp.where(s < 0, s ^ jnp.int32(0x7FFFFFFF), s)


def _body(k, x_ref, o_ref, key_ref):
    n, w = x_ref.shape
    key_ref[...] = _f32_sort_key(x_ref[...])

    def bit_step(i, t):
        cand = t ^ (jnp.int32(1) << (jnp.int32(31) - i))
        cnt = jnp.sum(
            (key_ref[...] >= cand).astype(jnp.int32), axis=0, keepdims=True
        )
        return jnp.where(cnt >= k, cand, t)

    t0 = jnp.full((1, w), jnp.iinfo(jnp.int32).min, dtype=jnp.int32)
    t = jax.lax.fori_loop(0, 32, bit_step, t0)
    o_ref[...] = jnp.where(key_ref[...] >= t, x_ref[...], jnp.float32(0.0))


@jax.jit
def kernel(x):
    n, d = x.shape
    k = min(_TOPK, n)
    w = 128
    grid = d // w
    return pl.pallas_call(
        functools.partial(_body, k),
        grid=(grid,),
        in_specs=[pl.BlockSpec((n, w), lambda i: (0, i))],
        out_specs=pl.BlockSpec((n, w), lambda i: (0, i)),
        out_shape=jax.ShapeDtypeStruct((n, d), jnp.float32),
        scratch_shapes=[pltpu.VMEM((n, w), jnp.int32)],
    )(x)


# speedup vs baseline: 36.6260x; 3.0415x over previous
import functools

import jax
import jax.numpy as jnp
import numpy as np
from jax.experimental import pallas as pl
from jax.experimental.pallas import tpu as pltpu

_TOPK = 256


def _f32_sort_key(x):
    s = jax.lax.bitcast_convert_type(x, jnp.int32)
    return jnp.where(s < 0, s ^ jnp.int32(0x7FFFFFFF), s)


def _body(k, x_ref, o_ref, key_ref):
    n, w = x_ref.shape
    key_ref[...] = _f32_sort_key(x_ref[...])
    ones = jnp.ones((8, n), dtype=jnp.float32)

    def bit_step(i, t):
        cand = t ^ (jnp.int32(1) << (jnp.int32(31) - i))
        mask = jnp.where(key_ref[...] >= cand, jnp.float32(1.0), jnp.float32(0.0))
        cnt = jax.lax.dot_general(
            ones, mask, (((1,), (0,)), ((), ())),
            preferred_element_type=jnp.float32,
        )[0:1, :]
        return jnp.where(cnt >= jnp.float32(k), cand, t)

    t0 = jnp.full((1, w), jnp.iinfo(jnp.int32).min, dtype=jnp.int32)
    t = jax.lax.fori_loop(0, 32, bit_step, t0)
    o_ref[...] = jnp.where(key_ref[...] >= t, x_ref[...], jnp.float32(0.0))


@jax.jit
def kernel(x):
    n, d = x.shape
    k = min(_TOPK, n)
    w = 128
    grid = d // w
    return pl.pallas_call(
        functools.partial(_body, k),
        grid=(grid,),
        in_specs=[pl.BlockSpec((n, w), lambda i: (0, i))],
        out_specs=pl.BlockSpec((n, w), lambda i: (0, i)),
        out_shape=jax.ShapeDtypeStruct((n, d), jnp.float32),
        scratch_shapes=[pltpu.VMEM((n, w), jnp.int32)],
    )(x)
